# baseline (speedup 1.0000x reference)
import functools

import jax
import jax.numpy as jnp
from jax import lax
from jax.experimental import pallas as pl
from jax.experimental.pallas import tpu as pltpu
from jax.experimental.pallas import tpu_sc as plsc

_NUM_FIELDS = 26
_VOCAB = 100000
_DIM = 32
_BATCH = 16384

_ALEN = 50048
_BOFF = _ALEN
_BLEN = 99968 - _ALEN
_TOFF = 99968
_TLEN = _VOCAB - _TOFF

_NC = 2
_NS = 16
_L = 16

_NW = _NC * _NS
_BPT = _BATCH // _NW
_SLAB = 128
_NROWS = _NUM_FIELDS * _DIM
_RPT = _NROWS // _NW
_CH = _BATCH // 2


def _body(idx_hbm, table_hbm, tail_hbm, fixed_hbm, out_hbm,
          rba, rbb, rbt, ibuf, obuf, slab_v, part_v, tot_v, fixed_v, sums2,
          shared, sa, sb, st, ws):
    c = lax.axis_index("c")
    s = lax.axis_index("s")
    wid = s * _NC + c
    b0 = wid * _BPT
    ob0 = (s * _NC + (1 - c)) * _BPT

    pltpu.sync_copy(fixed_hbm, fixed_v)

    def sum_slab(col0, init):
        pltpu.sync_copy(
            idx_hbm.at[pl.ds(0, _NUM_FIELDS), pl.ds(col0, _SLAB)], slab_v)

        def per_field(f, carry):
            acc = jnp.zeros((_L,), jnp.int32) if init else \
                part_v[pl.ds(f * _L, _L)]
            for u in range(_SLAB // _L):
                acc = acc + slab_v[f, pl.ds(u * _L, _L)]
            part_v[pl.ds(f * _L, _L)] = acc
            return carry

        lax.fori_loop(0, _NUM_FIELDS, per_field, 0)

    for k in range(_BPT // _SLAB):
        sum_slab(ob0 + k * _SLAB, k == 0)
    for k in range(_BPT // _SLAB):
        sum_slab(b0 + k * _SLAB, False)

    pltpu.sync_copy(part_v, shared.at[s])
    plsc.subcore_barrier()

    def zero_tot(f, carry):
        tot_v[pl.ds(f * _L, _L)] = jnp.zeros((_L,), jnp.int32)
        return carry

    lax.fori_loop(0, _NUM_FIELDS, zero_tot, 0)
    for k in range(_NS):
        pltpu.sync_copy(shared.at[pl.ds(k, 1)], sums2)

        def add_chunk(f, carry):
            tot_v[pl.ds(f * _L, _L)] = (
                tot_v[pl.ds(f * _L, _L)] + sums2[0, pl.ds(f * _L, _L)])
            return carry

        lax.fori_loop(0, _NUM_FIELDS, add_chunk, 0)

    lanes = lax.iota(jnp.int32, _L)
    rbase = wid * _RPT

    def fire_spans(r):
        pltpu.make_async_copy(
            table_hbm.at[r].at[pl.ds(0, _ALEN)], rba, sa).start()
        pltpu.make_async_copy(
            table_hbm.at[r].at[pl.ds(_BOFF, _BLEN)], rbb, sb).start()
        pltpu.make_async_copy(tail_hbm.at[lax.div(r, 4)], rbt, st).start()

    def wait_spans(r):
        pltpu.make_async_copy(
            table_hbm.at[r].at[pl.ds(0, _ALEN)], rba, sa).wait()
        pltpu.make_async_copy(
            table_hbm.at[r].at[pl.ds(_BOFF, _BLEN)], rbb, sb).wait()
        pltpu.make_async_copy(tail_hbm.at[lax.div(r, 4)], rbt, st).wait()

    fire_spans(rbase)

    def wait_write():
        pltpu.make_async_copy(
            obuf, out_hbm.at[0, pl.ds(0, _CH)], ws).wait()

    def per_row(i, prev_f):
        r = rbase + i
        f = lax.div(r, _DIM)
        d = lax.rem(r, _DIM)

        @pl.when(f != prev_f)
        def _():
            pltpu.sync_copy(idx_hbm.at[f], ibuf)

        masked = jnp.sum(tot_v[pl.ds(f * _L, _L)]) == 0
        wait_spans(r)

        for ch in range(2):
            if ch == 0:
                @pl.when(i > 0)
                def _():
                    wait_write()
            else:
                wait_write()

            @pl.when(jnp.logical_not(masked))
            def _():
                def ga8(j, carry3):
                    for u in range(8):
                        off = ch * _CH + (j * 8 + u) * _L
                        loc = (j * 8 + u) * _L
                        iv = ibuf[pl.ds(off, _L)]
                        m = iv < _ALEN
                        vals = plsc.load_gather(rba, [iv], mask=m)
                        plsc.store_scatter(obuf, [loc + lanes], vals, mask=m)
                    return carry3

                lax.fori_loop(0, _CH // (8 * _L), ga8, 0)

                def gb8(j, carry3):
                    for u in range(8):
                        off = ch * _CH + (j * 8 + u) * _L
                        loc = (j * 8 + u) * _L
                        iv = ibuf[pl.ds(off, _L)]
                        mb = (iv >= _ALEN) & (iv < _TOFF)
                        vals = plsc.load_gather(rbb, [iv - _BOFF], mask=mb)
                        plsc.store_scatter(obuf, [loc + lanes], vals, mask=mb)
                        mt = iv >= _TOFF
                        tv = plsc.load_gather(
                            rbt, [iv - _TOFF + lax.rem(r, 4) * _TLEN],
                            mask=mt)
                        plsc.store_scatter(obuf, [loc + lanes], tv, mask=mt)
                    return carry3

                lax.fori_loop(0, _CH // (8 * _L), gb8, 0)

            @pl.when(masked)
            def _():
                fv = fixed_v[pl.ds(d, _L)][0]
                splat = jnp.full((_L,), fv, jnp.float32)

                def fill(j, carry3):
                    for u in range(8):
                        obuf[pl.ds((j * 8 + u) * _L, _L)] = splat
                    return carry3

                lax.fori_loop(0, _CH // (8 * _L), fill, 0)

            pltpu.make_async_copy(
                obuf, out_hbm.at[r, pl.ds(ch * _CH, _CH)], ws).start()

        @pl.when(i < _RPT - 1)
        def _():
            fire_spans(r + 1)

        return f

    lax.fori_loop(0, _RPT, per_row, jnp.int32(-1))
    wait_write()


@functools.partial(
    pl.kernel,
    out_type=jax.ShapeDtypeStruct((_NROWS, _BATCH), jnp.float32),
    mesh=plsc.VectorSubcoreMesh(core_axis_name="c", subcore_axis_name="s"),
    compiler_params=pltpu.CompilerParams(
        needs_layout_passes=False, use_tc_tiling_on_sc=True),
    scratch_types=[
        pltpu.VMEM((_ALEN,), jnp.float32),
        pltpu.VMEM((_BLEN,), jnp.float32),
        pltpu.VMEM((128,), jnp.float32),
        pltpu.VMEM((_BATCH,), jnp.int32),
        pltpu.VMEM((_CH,), jnp.float32),
        pltpu.VMEM((_NUM_FIELDS, _SLAB), jnp.int32),
        pltpu.VMEM((_NUM_FIELDS * _L,), jnp.int32),
        pltpu.VMEM((_NUM_FIELDS * _L,), jnp.int32),
        pltpu.VMEM((_DIM + _L,), jnp.float32),
        pltpu.VMEM((1, _NUM_FIELDS * _L), jnp.int32),
        pltpu.VMEM_SHARED((_NS, _NUM_FIELDS * _L), jnp.int32),
        pltpu.SemaphoreType.DMA,
        pltpu.SemaphoreType.DMA,
        pltpu.SemaphoreType.DMA,
        pltpu.SemaphoreType.DMA,
    ],
)
def _sc_embedding(idx_hbm, table_hbm, tail_hbm, fixed_hbm, out_hbm, *scratch):
    _body(idx_hbm, table_hbm, tail_hbm, fixed_hbm, out_hbm, *scratch)


def kernel(sparse_inputs, tables, fixed_vector):
    idx_t = sparse_inputs.astype(jnp.int32).T
    table_t = tables.transpose(0, 2, 1).reshape(_NROWS, _VOCAB)
    fixed = jnp.pad(fixed_vector.astype(jnp.float32), (0, _L))
    tails = tables[:, _TOFF:, :].transpose(0, 2, 1).reshape(_NROWS // 4, 128)
    out2 = _sc_embedding(idx_t, table_t, tails, fixed)
    return out2.reshape(_NUM_FIELDS, _DIM, _BATCH).transpose(2, 0, 1)
